# baseline (speedup 1.0000x reference)
import functools

import jax
import jax.numpy as jnp
from jax import lax
from jax.experimental import pallas as pl
from jax.experimental.pallas import tpu as pltpu
from jax.experimental.pallas import tpu_sc as plsc

D_MODEL = 768
SEQ_LEN = 2048
BATCH = 4

NUM_WORKERS = 32
T_PER_W = SEQ_LEN // NUM_WORKERS
SUB = 32
H = T_PER_W // SUB
NSUB = BATCH * H
VECS = D_MODEL // 16

_mesh = plsc.VectorSubcoreMesh(
    core_axis_name="c", subcore_axis_name="s", num_cores=2, num_subcores=16
)


@functools.partial(
    pl.kernel,
    out_type=jax.ShapeDtypeStruct((BATCH * SEQ_LEN, D_MODEL), jnp.float32),
    mesh=_mesh,
    scratch_types=[
        pltpu.VMEM((BATCH, T_PER_W), jnp.int32),
        pltpu.VMEM((T_PER_W, D_MODEL), jnp.float32),
        pltpu.VMEM((SUB, D_MODEL), jnp.float32),
        pltpu.VMEM((SUB, D_MODEL), jnp.float32),
        pltpu.SemaphoreType.DMA,
        pltpu.SemaphoreType.DMA,
        pltpu.SemaphoreType.DMA,
        pltpu.SemaphoreType.DMA,
        pltpu.SemaphoreType.DMA,
    ],
)
def _embed_sc(idx_hbm, w_hbm, pos_hbm, out_hbm, idx_v, pos_v, buf0, buf1,
              pos_sem, g_sem0, g_sem1, st_sem0, st_sem1):
    wid = lax.axis_index("s") * 2 + lax.axis_index("c")
    t0 = wid * T_PER_W

    for b in range(BATCH):
        pltpu.sync_copy(idx_hbm.at[pl.ds(b * SEQ_LEN + t0, T_PER_W)],
                        idx_v.at[b])
    pos_desc = pltpu.async_copy(pos_hbm.at[pl.ds(t0, T_PER_W)], pos_v, pos_sem)

    bufs = (buf0, buf1)
    g_sems = (g_sem0, g_sem1)
    st_sems = (st_sem0, st_sem1)

    def gather(k):
        b, h = divmod(k, H)
        return pltpu.async_copy(
            w_hbm.at[idx_v.at[b, pl.ds(h * SUB, SUB)]], bufs[k % 2],
            g_sems[k % 2])

    def out_slice(k):
        b, h = divmod(k, H)
        return out_hbm.at[pl.ds(b * SEQ_LEN + t0 + h * SUB, SUB)]

    g_descs = [None] * NSUB
    st_descs = [None] * NSUB
    g_descs[0] = gather(0)
    for k in range(NSUB):
        buf = bufs[k % 2]
        g_descs[k].wait()
        if k + 1 < NSUB:
            if k >= 1:
                st_descs[k - 1].wait()
            g_descs[k + 1] = gather(k + 1)
        if k == 0:
            pos_desc.wait()
        h = k % H

        @plsc.parallel_loop(0, SUB, unroll=4)
        def _add_row(r):
            for v in range(VECS):
                sl = pl.ds(v * 16, 16)
                plsc.addupdate(buf.at[r, sl], pos_v[h * SUB + r, sl])

        st_descs[k] = pltpu.async_copy(buf, out_slice(k), st_sems[k % 2])
    st_descs[NSUB - 2].wait()
    st_descs[NSUB - 1].wait()


def kernel(x, W, pos_embd):
    idx = x.astype(jnp.int32).reshape(BATCH * SEQ_LEN)
    out = _embed_sc(idx, W, pos_embd)
    return out.reshape(BATCH, SEQ_LEN, D_MODEL)


# speedup vs baseline: 1.0773x; 1.0773x over previous
import functools

import jax
import jax.numpy as jnp
from jax import lax
from jax.experimental import pallas as pl
from jax.experimental.pallas import tpu as pltpu
from jax.experimental.pallas import tpu_sc as plsc

D_MODEL = 768
SEQ_LEN = 2048
BATCH = 4

NUM_WORKERS = 32
T_PER_W = SEQ_LEN // NUM_WORKERS
SUB = 32
H = T_PER_W // SUB
NSUB = BATCH * H
VECS = D_MODEL // 16

_mesh = plsc.VectorSubcoreMesh(
    core_axis_name="c", subcore_axis_name="s", num_cores=2, num_subcores=16
)


@functools.partial(
    pl.kernel,
    out_type=jax.ShapeDtypeStruct((BATCH * SEQ_LEN, D_MODEL), jnp.float32),
    mesh=_mesh,
    scratch_types=[
        pltpu.VMEM((BATCH, T_PER_W), jnp.int32),
        pltpu.VMEM((T_PER_W, D_MODEL), jnp.float32),
        pltpu.VMEM((SUB, D_MODEL), jnp.float32),
        pltpu.VMEM((SUB, D_MODEL), jnp.float32),
        pltpu.VMEM((SUB, D_MODEL), jnp.float32),
        pltpu.SemaphoreType.DMA,
        pltpu.SemaphoreType.DMA,
        pltpu.SemaphoreType.DMA,
        pltpu.SemaphoreType.DMA,
        pltpu.SemaphoreType.DMA,
        pltpu.SemaphoreType.DMA,
        pltpu.SemaphoreType.DMA,
    ],
)
def _embed_sc(idx_hbm, w_hbm, pos_hbm, out_hbm, idx_v, pos_v, buf0, buf1,
              buf2, pos_sem, g_sem0, g_sem1, g_sem2, st_sem0, st_sem1,
              st_sem2):
    wid = lax.axis_index("s") * 2 + lax.axis_index("c")
    t0 = wid * T_PER_W

    for b in range(BATCH):
        pltpu.sync_copy(idx_hbm.at[pl.ds(b * SEQ_LEN + t0, T_PER_W)],
                        idx_v.at[b])
    pos_desc = pltpu.async_copy(pos_hbm.at[pl.ds(t0, T_PER_W)], pos_v, pos_sem)

    NBUF = 3
    bufs = (buf0, buf1, buf2)
    g_sems = (g_sem0, g_sem1, g_sem2)
    st_sems = (st_sem0, st_sem1, st_sem2)

    def gather(k):
        b, h = divmod(k, H)
        return pltpu.async_copy(
            w_hbm.at[idx_v.at[b, pl.ds(h * SUB, SUB)]], bufs[k % NBUF],
            g_sems[k % NBUF])

    def out_slice(k):
        b, h = divmod(k, H)
        return out_hbm.at[pl.ds(b * SEQ_LEN + t0 + h * SUB, SUB)]

    g_descs = [None] * NSUB
    st_descs = [None] * NSUB
    g_descs[0] = gather(0)
    g_descs[1] = gather(1)
    for k in range(NSUB):
        buf = bufs[k % NBUF]
        g_descs[k].wait()
        if k + 2 < NSUB:
            if k >= 1:
                st_descs[k - 1].wait()
            g_descs[k + 2] = gather(k + 2)
        if k == 0:
            pos_desc.wait()
        h = k % H

        @plsc.parallel_loop(0, SUB, unroll=2)
        def _add_row(r):
            for v in range(VECS):
                sl = pl.ds(v * 16, 16)
                plsc.addupdate(buf.at[r, sl], pos_v[h * SUB + r, sl])

        st_descs[k] = pltpu.async_copy(buf, out_slice(k), st_sems[k % NBUF])
    for k in range(NSUB - NBUF, NSUB):
        st_descs[k].wait()


def kernel(x, W, pos_embd):
    idx = x.astype(jnp.int32).reshape(BATCH * SEQ_LEN)
    out = _embed_sc(idx, W, pos_embd)
    return out.reshape(BATCH, SEQ_LEN, D_MODEL)


# speedup vs baseline: 1.1198x; 1.0394x over previous
import functools

import jax
import jax.numpy as jnp
from jax import lax
from jax.experimental import pallas as pl
from jax.experimental.pallas import tpu as pltpu
from jax.experimental.pallas import tpu_sc as plsc

D_MODEL = 768
SEQ_LEN = 2048
BATCH = 4

NUM_WORKERS = 32
T_PER_W = SEQ_LEN // NUM_WORKERS
WIN = 16
NWIN = T_PER_W // WIN
VECS = D_MODEL // 16

_mesh = plsc.VectorSubcoreMesh(
    core_axis_name="c", subcore_axis_name="s", num_cores=2, num_subcores=16
)

_BUF = pltpu.VMEM((WIN, D_MODEL), jnp.float32)


@functools.partial(
    pl.kernel,
    out_type=jax.ShapeDtypeStruct((BATCH * SEQ_LEN, D_MODEL), jnp.float32),
    mesh=_mesh,
    scratch_types=[
        pltpu.VMEM((BATCH, T_PER_W), jnp.int32),
        [[_BUF for _ in range(BATCH)] for _ in range(2)],
        [_BUF, _BUF],
        [pltpu.SemaphoreType.DMA for _ in range(2)],
        [pltpu.SemaphoreType.DMA for _ in range(2)],
    ],
)
def _embed_sc(idx_hbm, w_hbm, pos_hbm, out_hbm, idx_v, gbufs, pbufs,
              ld_sems, st_sems):
    wid = lax.axis_index("s") * 2 + lax.axis_index("c")
    t0 = wid * T_PER_W

    for b in range(BATCH):
        pltpu.sync_copy(idx_hbm.at[b, pl.ds(t0, T_PER_W)], idx_v.at[b])

    def start_loads(w):
        p = w % 2
        descs = [pltpu.async_copy(
            w_hbm.at[idx_v.at[b, pl.ds(w * WIN, WIN)]], gbufs[p][b],
            ld_sems[p]) for b in range(BATCH)]
        descs.append(pltpu.async_copy(
            pos_hbm.at[pl.ds(t0 + w * WIN, WIN)], pbufs[p], ld_sems[p]))
        return descs

    def start_stores(w):
        p = w % 2
        return [pltpu.async_copy(
            gbufs[p][b],
            out_hbm.at[pl.ds(b * SEQ_LEN + t0 + w * WIN, WIN)],
            st_sems[p]) for b in range(BATCH)]

    ld_descs = [None] * NWIN
    st_descs = [None] * NWIN
    ld_descs[0] = start_loads(0)
    for w in range(NWIN):
        p = w % 2
        if w + 1 < NWIN:
            if w >= 1:
                for d in st_descs[w - 1]:
                    d.wait()
            ld_descs[w + 1] = start_loads(w + 1)
        for d in ld_descs[w]:
            d.wait()
        gb = gbufs[p]
        pb = pbufs[p]

        @plsc.parallel_loop(0, WIN, unroll=2)
        def _add_row(r):
            for v in range(VECS):
                sl = pl.ds(v * 16, 16)
                pvec = pb[r, sl]
                for b in range(BATCH):
                    plsc.addupdate(gb[b].at[r, sl], pvec)

        st_descs[w] = start_stores(w)
    for w in (NWIN - 2, NWIN - 1):
        for d in st_descs[w]:
            d.wait()


def kernel(x, W, pos_embd):
    idx = x if x.dtype == jnp.int32 else x.astype(jnp.int32)
    out = _embed_sc(idx, W, pos_embd)
    return out.reshape(BATCH, SEQ_LEN, D_MODEL)


# speedup vs baseline: 1.1491x; 1.0262x over previous
import functools

import jax
import jax.numpy as jnp
from jax import lax
from jax.experimental import pallas as pl
from jax.experimental.pallas import tpu as pltpu
from jax.experimental.pallas import tpu_sc as plsc

D_MODEL = 768
SEQ_LEN = 2048
BATCH = 4

NUM_WORKERS = 32
T_PER_W = SEQ_LEN // NUM_WORKERS
WIN = 16
NWIN = T_PER_W // WIN
VECS = D_MODEL // 16

_mesh = plsc.VectorSubcoreMesh(
    core_axis_name="c", subcore_axis_name="s", num_cores=2, num_subcores=16
)

_BUF = pltpu.VMEM((WIN, D_MODEL), jnp.float32)


@functools.partial(
    pl.kernel,
    out_type=jax.ShapeDtypeStruct((BATCH * SEQ_LEN, D_MODEL), jnp.float32),
    mesh=_mesh,
    scratch_types=[
        pltpu.VMEM((BATCH, T_PER_W), jnp.int32),
        [[_BUF for _ in range(BATCH)] for _ in range(2)],
        [_BUF, _BUF],
        [pltpu.SemaphoreType.DMA for _ in range(2)],
        [pltpu.SemaphoreType.DMA for _ in range(2)],
    ],
)
def _embed_sc(idx_hbm, w_hbm, pos_hbm, out_hbm, idx_v, gbufs, pbufs,
              ld_sems, st_sems):
    wid = lax.axis_index("s") * 2 + lax.axis_index("c")
    t0 = wid * T_PER_W

    idx_descs = [
        pltpu.async_copy(idx_hbm.at[b, pl.ds(t0, T_PER_W)], idx_v.at[b],
                         ld_sems[0])
        for b in range(BATCH)]
    for d in idx_descs:
        d.wait()

    def start_loads(w):
        p = w % 2
        descs = [pltpu.async_copy(
            w_hbm.at[idx_v.at[b, pl.ds(w * WIN, WIN)]], gbufs[p][b],
            ld_sems[p]) for b in range(BATCH)]
        descs.append(pltpu.async_copy(
            pos_hbm.at[pl.ds(t0 + w * WIN, WIN)], pbufs[p], ld_sems[p]))
        return descs

    def start_stores(w):
        p = w % 2
        return [pltpu.async_copy(
            gbufs[p][b],
            out_hbm.at[pl.ds(b * SEQ_LEN + t0 + w * WIN, WIN)],
            st_sems[p]) for b in range(BATCH)]

    ld_descs = [None] * NWIN
    st_descs = [None] * NWIN
    ld_descs[0] = start_loads(0)
    for w in range(NWIN):
        p = w % 2
        if w + 1 < NWIN:
            if w >= 1:
                for d in st_descs[w - 1]:
                    d.wait()
            ld_descs[w + 1] = start_loads(w + 1)
        for d in ld_descs[w]:
            d.wait()
        gb = gbufs[p]
        pb = pbufs[p]

        @plsc.parallel_loop(0, WIN, unroll=2)
        def _add_row(r):
            for v in range(VECS):
                sl = pl.ds(v * 16, 16)
                pvec = pb[r, sl]
                for b in range(BATCH):
                    plsc.addupdate(gb[b].at[r, sl], pvec)

        st_descs[w] = start_stores(w)
    for w in (NWIN - 2, NWIN - 1):
        for d in st_descs[w]:
            d.wait()


def kernel(x, W, pos_embd):
    idx = x if x.dtype == jnp.int32 else x.astype(jnp.int32)
    out = _embed_sc(idx, W, pos_embd)
    return out.reshape(BATCH, SEQ_LEN, D_MODEL)


# speedup vs baseline: 1.2084x; 1.0516x over previous
import functools

import jax
import jax.numpy as jnp
from jax import lax
from jax.experimental import pallas as pl
from jax.experimental.pallas import tpu as pltpu
from jax.experimental.pallas import tpu_sc as plsc

D_MODEL = 768
SEQ_LEN = 2048
BATCH = 4

NUM_WORKERS = 32
T_PER_W = SEQ_LEN // NUM_WORKERS
WIN = 16
NWIN = T_PER_W // WIN
VECS = D_MODEL // 16

_mesh = plsc.VectorSubcoreMesh(
    core_axis_name="c", subcore_axis_name="s", num_cores=2, num_subcores=16
)

_BUF = pltpu.VMEM((WIN, D_MODEL), jnp.float32)


@functools.partial(
    pl.kernel,
    out_type=jax.ShapeDtypeStruct((BATCH * SEQ_LEN, D_MODEL), jnp.float32),
    mesh=_mesh,
    scratch_types=[
        pltpu.VMEM((BATCH, T_PER_W), jnp.int32),
        [[_BUF for _ in range(BATCH)] for _ in range(2)],
        [_BUF, _BUF],
        [pltpu.SemaphoreType.DMA for _ in range(2)],
        [pltpu.SemaphoreType.DMA for _ in range(2)],
    ],
)
def _embed_sc(idx_hbm, w_hbm, pos_hbm, out_hbm, idx_v, gbufs, pbufs,
              ld_sems, st_sems):
    wid = lax.axis_index("s") * 2 + lax.axis_index("c")
    t0 = wid * T_PER_W

    idx_descs = [
        pltpu.async_copy(idx_hbm.at[b, pl.ds(t0, T_PER_W)], idx_v.at[b],
                         ld_sems[0])
        for b in range(BATCH)]
    for d in idx_descs:
        d.wait()

    def start_loads(w):
        p = w % 2
        descs = [pltpu.async_copy(
            w_hbm.at[idx_v.at[b, pl.ds(w * WIN, WIN)]], gbufs[p][b],
            ld_sems[p]) for b in range(BATCH)]
        descs.append(pltpu.async_copy(
            pos_hbm.at[pl.ds(t0 + w * WIN, WIN)], pbufs[p], ld_sems[p]))
        return descs

    def start_stores(w):
        p = w % 2
        return [pltpu.async_copy(
            gbufs[p][b],
            out_hbm.at[pl.ds(b * SEQ_LEN + t0 + w * WIN, WIN)],
            st_sems[p]) for b in range(BATCH)]

    ld_descs = [None] * NWIN
    st_descs = [None] * NWIN
    ld_descs[0] = start_loads(0)
    for w in range(NWIN):
        p = w % 2
        if w + 1 < NWIN:
            if w >= 1:
                for d in st_descs[w - 1]:
                    d.wait()
            ld_descs[w + 1] = start_loads(w + 1)
        for d in ld_descs[w]:
            d.wait()
        gb = gbufs[p]
        pb = pbufs[p]

        @plsc.parallel_loop(0, WIN, unroll=1)
        def _add_row(r):
            for v in range(VECS):
                sl = pl.ds(v * 16, 16)
                pvec = pb[r, sl]
                for b in range(BATCH):
                    plsc.addupdate(gb[b].at[r, sl], pvec)

        st_descs[w] = start_stores(w)
    for w in (NWIN - 2, NWIN - 1):
        for d in st_descs[w]:
            d.wait()


def kernel(x, W, pos_embd):
    idx = x if x.dtype == jnp.int32 else x.astype(jnp.int32)
    out = _embed_sc(idx, W, pos_embd)
    return out.reshape(BATCH, SEQ_LEN, D_MODEL)


# speedup vs baseline: 1.3001x; 1.0758x over previous
import functools

import jax
import jax.numpy as jnp
from jax import lax
from jax.experimental import pallas as pl
from jax.experimental.pallas import tpu as pltpu
from jax.experimental.pallas import tpu_sc as plsc

D_MODEL = 768
SEQ_LEN = 2048
BATCH = 4

NUM_WORKERS = 32
T_PER_W = SEQ_LEN // NUM_WORKERS
WIN = 16
NWIN = T_PER_W // WIN
VECS = D_MODEL // 16

_mesh = plsc.VectorSubcoreMesh(
    core_axis_name="c", subcore_axis_name="s", num_cores=2, num_subcores=16
)

_BUF = pltpu.VMEM((WIN, D_MODEL), jnp.float32)


@functools.partial(
    pl.kernel,
    out_type=jax.ShapeDtypeStruct((BATCH * SEQ_LEN, D_MODEL), jnp.float32),
    mesh=_mesh,
    scratch_types=[
        pltpu.VMEM((BATCH, T_PER_W), jnp.int32),
        [[_BUF for _ in range(BATCH)] for _ in range(2)],
        [_BUF, _BUF],
        [pltpu.SemaphoreType.DMA for _ in range(2)],
        [pltpu.SemaphoreType.DMA for _ in range(2)],
    ],
)
def _embed_sc(idx_hbm, w_hbm, pos_hbm, out_hbm, idx_v, gbufs, pbufs,
              ld_sems, st_sems):
    wid = lax.axis_index("s") * 2 + lax.axis_index("c")
    t0 = wid * T_PER_W

    idx_descs = [
        pltpu.async_copy(idx_hbm.at[b, pl.ds(t0, T_PER_W)], idx_v.at[b],
                         ld_sems[0])
        for b in range(BATCH)]
    for d in idx_descs:
        d.wait()

    def start_loads(w):
        p = w % 2
        descs = [pltpu.async_copy(
            w_hbm.at[idx_v.at[b, pl.ds(w * WIN, WIN)]], gbufs[p][b],
            ld_sems[p]) for b in range(BATCH)]
        descs.append(pltpu.async_copy(
            pos_hbm.at[pl.ds(t0 + w * WIN, WIN)], pbufs[p], ld_sems[p]))
        return descs

    def start_stores(w):
        p = w % 2
        return [pltpu.async_copy(
            gbufs[p][b],
            out_hbm.at[pl.ds(b * SEQ_LEN + t0 + w * WIN, WIN)],
            st_sems[p]) for b in range(BATCH)]

    ld_descs = [None] * NWIN
    st_descs = [None] * NWIN
    ld_descs[0] = start_loads(0)
    for w in range(NWIN):
        p = w % 2
        if w + 1 < NWIN:
            if w >= 1:
                for d in st_descs[w - 1]:
                    d.wait()
            ld_descs[w + 1] = start_loads(w + 1)
        for d in ld_descs[w]:
            d.wait()
        gb = gbufs[p]
        pb = pbufs[p]

        @plsc.parallel_loop(0, VECS, unroll=1)
        def _add_col(v):
            sl = pl.ds(v * 16, 16)
            for r in range(WIN):
                pvec = pb[r, sl]
                for b in range(BATCH):
                    plsc.addupdate(gb[b].at[r, sl], pvec)

        st_descs[w] = start_stores(w)
    for w in (NWIN - 2, NWIN - 1):
        for d in st_descs[w]:
            d.wait()


def kernel(x, W, pos_embd):
    idx = x if x.dtype == jnp.int32 else x.astype(jnp.int32)
    out = _embed_sc(idx, W, pos_embd)
    return out.reshape(BATCH, SEQ_LEN, D_MODEL)


# speedup vs baseline: 1.3368x; 1.0283x over previous
import functools

import jax
import jax.numpy as jnp
from jax import lax
from jax.experimental import pallas as pl
from jax.experimental.pallas import tpu as pltpu
from jax.experimental.pallas import tpu_sc as plsc

D_MODEL = 768
SEQ_LEN = 2048
BATCH = 4

NUM_WORKERS = 32
T_PER_W = SEQ_LEN // NUM_WORKERS
WIN = 16
NWIN = T_PER_W // WIN
VECS = D_MODEL // 16

_mesh = plsc.VectorSubcoreMesh(
    core_axis_name="c", subcore_axis_name="s", num_cores=2, num_subcores=16
)

_BUF = pltpu.VMEM((WIN, D_MODEL), jnp.float32)


@functools.partial(
    pl.kernel,
    out_type=jax.ShapeDtypeStruct((BATCH * SEQ_LEN, D_MODEL), jnp.float32),
    mesh=_mesh,
    scratch_types=[
        pltpu.VMEM((BATCH, T_PER_W), jnp.int32),
        [[_BUF for _ in range(BATCH)] for _ in range(2)],
        [_BUF, _BUF],
        [pltpu.SemaphoreType.DMA for _ in range(2)],
        [pltpu.SemaphoreType.DMA for _ in range(2)],
    ],
)
def _embed_sc(idx_hbm, w_hbm, pos_hbm, out_hbm, idx_v, gbufs, pbufs,
              ld_sems, st_sems):
    wid = lax.axis_index("s") * 2 + lax.axis_index("c")
    t0 = wid * T_PER_W

    idx_descs = [
        pltpu.async_copy(idx_hbm.at[b, pl.ds(t0, T_PER_W)], idx_v.at[b],
                         ld_sems[0])
        for b in range(BATCH)]
    for d in idx_descs:
        d.wait()

    def loads(w, p):
        descs = [pltpu.make_async_copy(
            w_hbm.at[idx_v.at[b, pl.ds(w * WIN, WIN)]], gbufs[p][b],
            ld_sems[p]) for b in range(BATCH)]
        descs.append(pltpu.make_async_copy(
            pos_hbm.at[pl.ds(t0 + w * WIN, WIN)], pbufs[p], ld_sems[p]))
        return descs

    def stores(w, p):
        return [pltpu.make_async_copy(
            gbufs[p][b],
            out_hbm.at[pl.ds(b * SEQ_LEN + t0 + w * WIN, WIN)],
            st_sems[p]) for b in range(BATCH)]

    def add_window(p):
        gb = gbufs[p]
        pb = pbufs[p]

        @plsc.parallel_loop(0, VECS, unroll=1)
        def _add_col(v):
            sl = pl.ds(v * 16, 16)
            for r in range(WIN):
                pvec = pb[r, sl]
                for b in range(BATCH):
                    plsc.addupdate(gb[b].at[r, sl], pvec)

    def win_step(w, p):
        @pl.when(w + 1 < NWIN)
        def _prefetch():
            @pl.when(w >= 1)
            def _drain():
                for d in stores(w - 1, 1 - p):
                    d.wait()
            for d in loads(w + 1, 1 - p):
                d.start()
        for d in loads(w, p):
            d.wait()
        add_window(p)
        for d in stores(w, p):
            d.start()

    for d in loads(0, 0):
        d.start()

    @pl.loop(0, NWIN // 2)
    def _pair(j):
        w0 = 2 * j
        win_step(w0, 0)
        win_step(w0 + 1, 1)

    for w, p in ((NWIN - 2, 0), (NWIN - 1, 1)):
        for d in stores(w, p):
            d.wait()


def kernel(x, W, pos_embd):
    idx = x if x.dtype == jnp.int32 else x.astype(jnp.int32)
    out = _embed_sc(idx, W, pos_embd)
    return out.reshape(BATCH, SEQ_LEN, D_MODEL)
